# baseline (speedup 1.0000x reference)
import jax
import jax.numpy as jnp
from jax.experimental import pallas as pl
from jax.experimental.pallas import tpu as pltpu

_BM1 = 400
_BM23 = 1000


def _b3(t):
    d = jnp.abs(t - 2.0)
    q = jnp.maximum(2.0 - d, 0.0)
    r = jnp.maximum(1.0 - d, 0.0)
    return (q * q * q - 4.0 * (r * r * r)) * (1.0 / 6.0)


def _feats(a):
    u = 2.0 * a + 5.0
    parts = [a * jax.nn.sigmoid(a)] + [_b3(u - j) for j in range(7)]
    return jnp.concatenate(parts, axis=1)


def _layer1_kernel(a_ref, v_ref, w_ref, h_ref, pre_ref, abf_ref):
    abf_ref[...] = a_ref[...].astype(jnp.bfloat16)
    a = jnp.dot(a_ref[...], v_ref[...], precision=jax.lax.Precision.DEFAULT,
                preferred_element_type=jnp.float32)
    pre_ref[...] = a
    h = jnp.dot(_feats(a), w_ref[...], preferred_element_type=jnp.float32)
    h_ref[...] = jnp.maximum(h, 0.0).astype(jnp.bfloat16)


def _layer_kernel(a_ref, v_ref, w_ref, h_ref, pre_ref, acc_ref):
    i = pl.program_id(0)
    s = jax.lax.rem(i, 2)
    acc_ref[s] = jnp.dot(a_ref[...], v_ref[...],
                         preferred_element_type=jnp.float32)

    @pl.when(i > 0)
    def _epilogue():
        a = acc_ref[1 - s]
        pre_ref[...] = a
        h = jnp.dot(_feats(a), w_ref[...], preferred_element_type=jnp.float32)
        h_ref[...] = jnp.maximum(h, 0.0).astype(jnp.bfloat16)


def _out_kernel(a_ref, v_ref, w_ref, p1_ref, p2_ref, o_ref, acc_ref):
    i = pl.program_id(0)
    s = jax.lax.rem(i, 2)
    acc_ref[s] = jnp.dot(a_ref[...], v_ref[...],
                         preferred_element_type=jnp.float32)

    @pl.when(i > 0)
    def _epilogue():
        o = jnp.dot(_feats(p1_ref[...]), w_ref[0],
                    preferred_element_type=jnp.float32)
        o += jnp.dot(_feats(p2_ref[...]), w_ref[1],
                     preferred_element_type=jnp.float32)
        o += jnp.dot(_feats(acc_ref[1 - s]), w_ref[2],
                     preferred_element_type=jnp.float32)
        o_ref[...] = jnp.maximum(o, 0.0)


def _layer1_call(adj, v, w):
    n, f = v.shape
    bm = _BM1
    nm = n // bm
    out_dim = w.shape[-1]
    return pl.pallas_call(
        _layer1_kernel,
        grid=(nm,),
        in_specs=[
            pl.BlockSpec((bm, n), lambda i: (i, 0)),
            pl.BlockSpec((n, f), lambda i: (0, 0)),
            pl.BlockSpec(w.shape, lambda i: (0, 0)),
        ],
        out_specs=[
            pl.BlockSpec((bm, out_dim), lambda i: (i, 0)),
            pl.BlockSpec((bm, f), lambda i: (i, 0)),
            pl.BlockSpec((bm, n), lambda i: (i, 0)),
        ],
        out_shape=[
            jax.ShapeDtypeStruct((n, out_dim), jnp.bfloat16),
            jax.ShapeDtypeStruct((n, f), jnp.float32),
            jax.ShapeDtypeStruct((n, n), jnp.bfloat16),
        ],
        compiler_params=pltpu.CompilerParams(
            dimension_semantics=("arbitrary",)),
    )(adj, v, w)


def _layer_call(adj, v, w):
    n, f = v.shape
    bm = _BM23
    nm = n // bm
    out_dim = w.shape[-1]
    last = nm - 1
    return pl.pallas_call(
        _layer_kernel,
        grid=(nm + 1,),
        in_specs=[
            pl.BlockSpec((bm, n), lambda i: (jnp.minimum(i, last), 0)),
            pl.BlockSpec((n, f), lambda i: (0, 0)),
            pl.BlockSpec(w.shape, lambda i: (0, 0)),
        ],
        out_specs=[
            pl.BlockSpec((bm, out_dim), lambda i: (jnp.maximum(i - 1, 0), 0)),
            pl.BlockSpec((bm, f), lambda i: (jnp.maximum(i - 1, 0), 0)),
        ],
        out_shape=[
            jax.ShapeDtypeStruct((n, out_dim), jnp.bfloat16),
            jax.ShapeDtypeStruct((n, f), jnp.float32),
        ],
        scratch_shapes=[pltpu.VMEM((2, bm, f), jnp.float32)],
        compiler_params=pltpu.CompilerParams(
            dimension_semantics=("arbitrary",)),
    )(adj, v, w)


def _out_call(adj, v, w, p1, p2):
    n, f = v.shape
    bm = _BM23
    nm = n // bm
    out_dim = w.shape[-1]
    last = nm - 1
    return pl.pallas_call(
        _out_kernel,
        grid=(nm + 1,),
        in_specs=[
            pl.BlockSpec((bm, n), lambda i: (jnp.minimum(i, last), 0)),
            pl.BlockSpec((n, f), lambda i: (0, 0)),
            pl.BlockSpec(w.shape, lambda i: (0, 0, 0)),
            pl.BlockSpec((bm, f), lambda i: (jnp.maximum(i - 1, 0), 0)),
            pl.BlockSpec((bm, f), lambda i: (jnp.maximum(i - 1, 0), 0)),
        ],
        out_specs=pl.BlockSpec((bm, out_dim),
                               lambda i: (jnp.maximum(i - 1, 0), 0)),
        out_shape=jax.ShapeDtypeStruct((n, out_dim), jnp.float32),
        scratch_shapes=[pltpu.VMEM((2, bm, f), jnp.float32)],
        compiler_params=pltpu.CompilerParams(
            dimension_semantics=("arbitrary",)),
    )(adj, v, w, p1, p2)


def _pack(base_w, spline_w, scaler):
    sw = spline_w * scaler[:, :, None]
    rows = [base_w.T] + [sw[:, :, j].T for j in range(7)]
    return jnp.concatenate(rows, axis=0)


def kernel(x, edge_index, base_w1, spline_w1, scaler1, base_w2, spline_w2,
           scaler2, base_wo, spline_wo, scaler_o):
    n, f = x.shape
    w1 = _pack(base_w1, spline_w1, scaler1)
    w2 = _pack(base_w2, spline_w2, scaler2)
    w3 = jnp.stack([
        _pack(base_wo[:, c * f:(c + 1) * f],
              spline_wo[:, c * f:(c + 1) * f],
              scaler_o[:, c * f:(c + 1) * f])
        for c in range(3)
    ])
    h, a1, adj_bf = _layer1_call(edge_index, x, w1)
    h2, a2 = _layer_call(adj_bf, h, w2)
    return _out_call(adj_bf, h2, w3, a1, a2)
